# Initial kernel scaffold; baseline (speedup 1.0000x reference)
import jax
import jax.numpy as jnp
from jax.experimental import pallas as pl


def kernel(obj_latents, wbig, consts):
    raise NotImplementedError("write your pallas kernel here")


# speedup vs baseline: 2.2223x; 2.2223x over previous
import math

import jax
import jax.numpy as jnp
from jax.experimental import pallas as pl
from jax.experimental.pallas import tpu as pltpu

_D_MODEL = 128
_N_HEADS = 8


def _make_body(d_model, n_heads, n_objs):
    rows_wo = ((n_heads + 7) // 8) * 8
    row_bbig = rows_wo + d_model
    row_bo = row_bbig + 1
    am_scale = math.sqrt(n_objs)

    def _body(x_ref, wbig_ref, consts_ref, out_ref, am_ref):
        bb = x_ref.shape[0]
        d_embed = x_ref.shape[2]
        n = bb * n_objs

        bbig = consts_ref[row_bbig:row_bbig + 1, :]
        x2 = x_ref[...].reshape(n, d_embed).astype(jnp.bfloat16)
        w = wbig_ref[...].astype(jnp.bfloat16)

        proj = jnp.dot(x2, w, preferred_element_type=jnp.float32) + bbig

        s_obj = proj[:, d_model:d_model + n_heads].reshape(bb, n_objs, n_heads)
        s_cls = bbig[:, d_model:d_model + n_heads]

        m = jnp.maximum(jnp.max(s_obj, axis=1), s_cls)
        e_obj = jnp.exp(s_obj - m[:, None, :])
        e_cls = jnp.exp(s_cls - m)
        inv = pl.reciprocal(jnp.sum(e_obj, axis=1) + e_cls, approx=True)
        p_obj = e_obj * inv[:, None, :]
        p_cls = e_cls * inv

        expand = consts_ref[0:n_heads, 0:d_model]
        e_exp = jnp.dot(p_obj.reshape(n, n_heads), expand,
                        preferred_element_type=jnp.float32)
        v_obj = proj[:, 0:d_model]
        ctx = jnp.sum((e_exp * v_obj).reshape(bb, n_objs, d_model), axis=1)

        bv = bbig[:, 0:d_model]
        ctx = ctx + jnp.dot(p_cls, expand * bv,
                            preferred_element_type=jnp.float32)

        wo = consts_ref[rows_wo:rows_wo + d_model, 0:d_model]
        bo = consts_ref[row_bo:row_bo + 1, 0:d_model]
        out_ref[...] = jnp.dot(ctx, wo, preferred_element_type=jnp.float32) + bo

        am_ref[...] = p_obj[:, :, 0] * am_scale

    return _body


def kernel(obj_latents, wbig, consts):
    bs, n_objs, d_embed = obj_latents.shape
    d_model, n_heads = _D_MODEL, _N_HEADS
    P = wbig.shape[1]
    Rc = consts.shape[0]
    f32 = jnp.float32

    n_blocks = 1
    for nb in (8, 4, 2):
        if bs % nb == 0:
            n_blocks = nb
            break
    bb = bs // n_blocks

    body = _make_body(d_model, n_heads, n_objs)
    rep = lambda b: (0, 0)
    out, am_all = pl.pallas_call(
        body,
        grid=(n_blocks,),
        in_specs=[
            pl.BlockSpec((bb, n_objs, d_embed), lambda b: (b, 0, 0)),
            pl.BlockSpec((d_embed, P), rep),
            pl.BlockSpec((Rc, P), rep),
        ],
        out_specs=[
            pl.BlockSpec((bb, d_model), lambda b: (b, 0)),
            pl.BlockSpec((bb, n_objs), lambda b: (b, 0)),
        ],
        out_shape=[
            jax.ShapeDtypeStruct((bs, d_model), f32),
            jax.ShapeDtypeStruct((bs, n_objs), f32),
        ],
        compiler_params=pltpu.CompilerParams(dimension_semantics=("parallel",)),
    )(obj_latents.astype(f32), wbig, consts)
    return out, am_all[0:1, :]


# speedup vs baseline: 2.3192x; 1.0436x over previous
import math

import jax
import jax.numpy as jnp
from jax.experimental import pallas as pl
from jax.experimental.pallas import tpu as pltpu

_D_MODEL = 128
_N_HEADS = 8


def _make_body(d_model, n_heads, n_objs):
    rows_wo = ((n_heads + 7) // 8) * 8
    row_bbig = rows_wo + d_model
    row_bo = row_bbig + 1
    am_scale = math.sqrt(n_objs)

    def _body(x_ref, wbig_ref, consts_ref, out_ref, amc_ref):
        bb = x_ref.shape[0]
        d_embed = x_ref.shape[2]
        n = bb * n_objs

        x2 = x_ref[...].reshape(n, d_embed).astype(jnp.bfloat16)
        w = wbig_ref[...].astype(jnp.bfloat16)

        proj = jnp.dot(x2, w, preferred_element_type=jnp.float32)

        e = jnp.exp(proj[:, d_model:d_model + n_heads])
        e3 = e.reshape(bb, n_objs, n_heads)
        inv = pl.reciprocal(jnp.sum(e3, axis=1) + 1.0, approx=True)
        p3 = e3 * inv[:, None, :]
        pf = p3.reshape(n, n_heads)

        expand = consts_ref[0:n_heads, 0:d_model]
        e_exp = jnp.dot(pf, expand, preferred_element_type=jnp.float32)
        y = e_exp * proj[:, 0:d_model]
        bv = consts_ref[row_bbig:row_bbig + 1, 0:d_model]
        ctx = jnp.sum(y.reshape(bb, n_objs, d_model), axis=1) + bv

        wo = consts_ref[rows_wo:rows_wo + d_model, 0:d_model]
        bo = consts_ref[row_bo:row_bo + 1, 0:d_model]
        out_ref[...] = jnp.dot(ctx, wo, preferred_element_type=jnp.float32) + bo

        amc_ref[...] = pf[:, 0:1] * am_scale

    return _body


def kernel(obj_latents, wbig, consts):
    bs, n_objs, d_embed = obj_latents.shape
    d_model, n_heads = _D_MODEL, _N_HEADS
    P = wbig.shape[1]
    Rc = consts.shape[0]
    f32 = jnp.float32

    n_blocks = 1
    for nb in (8, 4, 2):
        if bs % nb == 0:
            n_blocks = nb
            break
    bb = bs // n_blocks

    body = _make_body(d_model, n_heads, n_objs)
    rep = lambda b: (0, 0)
    out, am_col = pl.pallas_call(
        body,
        grid=(n_blocks,),
        in_specs=[
            pl.BlockSpec((bb, n_objs, d_embed), lambda b: (b, 0, 0)),
            pl.BlockSpec((d_embed, P), rep),
            pl.BlockSpec((Rc, P), rep),
        ],
        out_specs=[
            pl.BlockSpec((bb, d_model), lambda b: (b, 0)),
            pl.BlockSpec((bb * n_objs, 1), lambda b: (b, 0)),
        ],
        out_shape=[
            jax.ShapeDtypeStruct((bs, d_model), f32),
            jax.ShapeDtypeStruct((bs * n_objs, 1), f32),
        ],
        compiler_params=pltpu.CompilerParams(dimension_semantics=("parallel",)),
    )(obj_latents.astype(f32), wbig, consts)
    return out, am_col.reshape(bs, n_objs)[0:1, :]
